# baseline (speedup 1.0000x reference)
import functools

import jax
import jax.numpy as jnp
import numpy as np
from jax.experimental import pallas as pl

V = 128
D = 64
DD = D * D
DERTA = 10.0

_Pa = np.zeros((D, DD), dtype=np.float32)
_Pb = np.zeros((D, DD), dtype=np.float32)
for _a in range(D):
    _Pa[_a, _a * D:(_a + 1) * D] = 1.0
for _b in range(D):
    _Pb[_b, _b::D] = 1.0
_PA = jnp.asarray(_Pa)
_PB = jnp.asarray(_Pb)

BI = 2


def _spc_kernel(cos_ref, feat_ref, pa_ref, pb_ref, eij_ref, sij_ref, c_ref):
    i = pl.program_id(0)

    @pl.when(i == 0)
    def _():
        eij = 0.5 * (1.0 - cos_ref[...])
        eij_ref[...] = eij
        sij_ref[...] = jnp.exp(-DERTA * eij)

    feats = feat_ref[...]
    fis = feat_ref[pl.ds(i * BI, BI), :]
    a_blk = jnp.dot(fis, pa_ref[...], preferred_element_type=jnp.float32)
    b_full = jnp.dot(feats, pb_ref[...], preferred_element_type=jnp.float32)
    c_ref[...] = a_blk[:, None, :] * b_full[None, :, :]


@jax.jit
def kernel(costheta, features):
    eij, sij, c_flat = pl.pallas_call(
        _spc_kernel,
        grid=(V // BI,),
        in_specs=[
            pl.BlockSpec((V, V), lambda i: (0, 0)),
            pl.BlockSpec((V, D), lambda i: (0, 0)),
            pl.BlockSpec((D, DD), lambda i: (0, 0)),
            pl.BlockSpec((D, DD), lambda i: (0, 0)),
        ],
        out_specs=[
            pl.BlockSpec((V, V), lambda i: (0, 0)),
            pl.BlockSpec((V, V), lambda i: (0, 0)),
            pl.BlockSpec((BI, V, DD), lambda i: (i, 0, 0)),
        ],
        out_shape=[
            jax.ShapeDtypeStruct((V, V), jnp.float32),
            jax.ShapeDtypeStruct((V, V), jnp.float32),
            jax.ShapeDtypeStruct((V, V, DD), jnp.float32),
        ],
    )(costheta, features, _PA, _PB)
    return (eij, sij, c_flat.reshape(V, V, D, D))
